# Initial kernel scaffold; baseline (speedup 1.0000x reference)
import jax
import jax.numpy as jnp
from jax.experimental import pallas as pl


def kernel(x):
    raise NotImplementedError("write your pallas kernel here")


# speedup vs baseline: 22.9741x; 22.9741x over previous
import jax
import jax.numpy as jnp
import numpy as np
from jax.experimental import pallas as pl

T_STEPS = 32


def _body(x_ref, o_ref):
    xb = x_ref[...]
    xc = jnp.clip(xb, 0.0, 1.0)
    t = (T_STEPS * (1.0 - xc)).astype(jnp.int32)
    t = jnp.clip(t, 0, T_STEPS - 1)
    S = xb.shape[1]
    tio = jax.lax.broadcasted_iota(jnp.int32, (1, T_STEPS, S, 128), 1)
    o_ref[...] = (tio == t[:, None, :, :]).astype(jnp.float32)


def kernel(x):
    B = x.shape[0]
    rest = x.shape[1:]
    F = int(np.prod(rest))
    assert F % 128 == 0
    Fs = F // 128
    S = 168 if Fs % 168 == 0 else 8
    assert Fs % S == 0
    x2 = x.reshape(B, Fs, 128)
    out = pl.pallas_call(
        _body,
        grid=(B, Fs // S),
        in_specs=[pl.BlockSpec((1, S, 128), lambda b, j: (b, j, 0))],
        out_specs=pl.BlockSpec((1, T_STEPS, S, 128), lambda b, j: (b, 0, j, 0)),
        out_shape=jax.ShapeDtypeStruct((B, T_STEPS, Fs, 128), jnp.float32),
    )(x2)
    return out.reshape((B, T_STEPS) + tuple(rest))


# speedup vs baseline: 23.6422x; 1.0291x over previous
import jax
import jax.numpy as jnp
import numpy as np
from jax.experimental import pallas as pl

T_STEPS = 32
T_CHUNK = 8


def _body(x_ref, o_ref):
    xb = x_ref[...]
    xc = jnp.clip(xb, 0.0, 1.0)
    t = (T_STEPS * (1.0 - xc)).astype(jnp.int32)
    t = jnp.clip(t, 0, T_STEPS - 1)
    Fs = xb.shape[1]
    t_base = pl.program_id(1) * T_CHUNK
    tio = t_base + jax.lax.broadcasted_iota(jnp.int32, (1, T_CHUNK, Fs, 128), 1)
    o_ref[...] = (tio == t[:, None, :, :]).astype(jnp.float32)


def kernel(x):
    B = x.shape[0]
    rest = x.shape[1:]
    F = int(np.prod(rest))
    assert F % 128 == 0
    Fs = F // 128
    x2 = x.reshape(B, Fs, 128)
    out = pl.pallas_call(
        _body,
        grid=(B, T_STEPS // T_CHUNK),
        in_specs=[pl.BlockSpec((1, Fs, 128), lambda b, tc: (b, 0, 0))],
        out_specs=pl.BlockSpec((1, T_CHUNK, Fs, 128), lambda b, tc: (b, tc, 0, 0)),
        out_shape=jax.ShapeDtypeStruct((B, T_STEPS, Fs, 128), jnp.float32),
    )(x2)
    return out.reshape((B, T_STEPS) + tuple(rest))
